# baseline (speedup 1.0000x reference)
import functools

import jax
import jax.numpy as jnp
from jax import lax
from jax.experimental import pallas as pl
from jax.experimental.pallas import tpu as pltpu
from jax.experimental.pallas import tpu_sc as plsc

_D, _H, _W = 8, 16, 16
_N = _D * _H * _W
_HW = _H * _W


def _slabs_body(bias_ref, out_ref):
    hw_r = jax.lax.broadcasted_iota(jnp.int32, (_HW, _HW), 0)
    hw_c = jax.lax.broadcasted_iota(jnp.int32, (_HW, _HW), 1)
    rh = (hw_r >> 4) - (hw_c >> 4)
    rw = (hw_r & 15) - (hw_c & 15)
    p2 = rh * rh + rw * rw
    t = [bias_ref[k, 0] for k in range(6)]
    for delta in range(15):
        s = (p2 + (delta - 7) * (delta - 7)).astype(jnp.float32)
        b = jnp.floor(jnp.sqrt(s) * 0.25)
        out_ref[delta] = jnp.where(
            b < 1.0, t[0],
            jnp.where(b < 2.0, t[1],
                      jnp.where(b < 3.0, t[2],
                                jnp.where(b < 4.0, t[3],
                                          jnp.where(b < 5.0, t[4], t[5])))))


def _build_slabs(rel_pos_bias):
    return pl.pallas_call(
        _slabs_body,
        in_specs=[pl.BlockSpec(memory_space=pltpu.VMEM)],
        out_specs=pl.BlockSpec(memory_space=pltpu.VMEM),
        out_shape=jax.ShapeDtypeStruct((15, _HW, _HW), jnp.float32),
    )(rel_pos_bias)


_mesh = plsc.VectorSubcoreMesh(core_axis_name="c", subcore_axis_name="s")


@functools.partial(
    pl.kernel,
    mesh=_mesh,
    out_type=jax.ShapeDtypeStruct((_N, _N), jnp.float32),
    scratch_types=[pltpu.SemaphoreType.DMA],
)
def _expand(slabs_hbm, out_hbm, sem):
    wid = lax.axis_index("s") * 2 + lax.axis_index("c")
    copies = []
    for k in range(2):
        blk = wid * 2 + k
        di = blk >> 3
        dj = blk & 7
        cp = pltpu.make_async_copy(
            slabs_hbm.at[di - dj + 7],
            out_hbm.at[pl.ds(di * _HW, _HW), pl.ds(dj * _HW, _HW)],
            sem)
        cp.start()
        copies.append(cp)
    for cp in copies:
        cp.wait()


def kernel(D, H, W, rel_pos_bias):
    del D, H, W
    return _expand(_build_slabs(rel_pos_bias))


# speedup vs baseline: 16.5332x; 16.5332x over previous
import functools

import jax
import jax.numpy as jnp
from jax import lax
from jax.experimental import pallas as pl
from jax.experimental.pallas import tpu as pltpu
from jax.experimental.pallas import tpu_sc as plsc

_D, _H, _W = 8, 16, 16
_N = _D * _H * _W
_HW = _H * _W
_NSLAB = 2 * _D - 1
_RPT = _HW // 32


def _slabs_body(bias_ref, out_ref):
    hw_r = jax.lax.broadcasted_iota(jnp.int32, (_HW, _HW), 0)
    hw_c = jax.lax.broadcasted_iota(jnp.int32, (_HW, _HW), 1)
    rh = (hw_r >> 4) - (hw_c >> 4)
    rw = (hw_r & 15) - (hw_c & 15)
    p2 = rh * rh + rw * rw
    t = [bias_ref[k, 0] for k in range(6)]
    for p in range(_NSLAB):
        s = (p2 + (7 - p) * (7 - p)).astype(jnp.float32)
        b = jnp.floor(jnp.sqrt(s) * 0.25)
        out_ref[p] = jnp.where(
            b < 1.0, t[0],
            jnp.where(b < 2.0, t[1],
                      jnp.where(b < 3.0, t[2],
                                jnp.where(b < 4.0, t[3],
                                          jnp.where(b < 5.0, t[4], t[5])))))


def _build_slabs(rel_pos_bias):
    return pl.pallas_call(
        _slabs_body,
        in_specs=[pl.BlockSpec(memory_space=pltpu.VMEM)],
        out_specs=pl.BlockSpec(memory_space=pltpu.VMEM),
        out_shape=jax.ShapeDtypeStruct((_NSLAB, _HW, _HW), jnp.float32),
    )(rel_pos_bias)


_mesh = plsc.VectorSubcoreMesh(core_axis_name="c", subcore_axis_name="s")


@functools.partial(
    pl.kernel,
    mesh=_mesh,
    out_type=jax.ShapeDtypeStruct((_N, _N), jnp.float32),
    scratch_types=[
        pltpu.VMEM((_RPT, _NSLAB * _HW), jnp.float32),
        pltpu.SemaphoreType.DMA,
        pltpu.SemaphoreType.DMA,
    ],
)
def _expand(slabs_hbm, out_hbm, buf, sem_in, sem_out):
    wid = lax.axis_index("s") * 2 + lax.axis_index("c")
    hw0 = wid * _RPT
    stages = []
    for p in range(_NSLAB):
        cp = pltpu.make_async_copy(
            slabs_hbm.at[p, pl.ds(hw0, _RPT), :],
            buf.at[:, pl.ds(p * _HW, _HW)],
            sem_in)
        cp.start()
        stages.append(cp)
    for cp in stages:
        cp.wait()
    writes = []
    for di in range(_D):
        for k in range(_RPT):
            i = di * _HW + hw0 + k
            cp = pltpu.make_async_copy(
                buf.at[k, pl.ds((7 - di) * _HW, _N)],
                out_hbm.at[i],
                sem_out)
            cp.start()
            writes.append(cp)
    for cp in writes:
        cp.wait()


def kernel(D, H, W, rel_pos_bias):
    del D, H, W
    return _expand(_build_slabs(rel_pos_bias))


# speedup vs baseline: 62.9587x; 3.8080x over previous
import jax
import jax.numpy as jnp
from jax.experimental import pallas as pl
from jax.experimental.pallas import tpu as pltpu

_D, _H, _W = 8, 16, 16
_N = _D * _H * _W
_HW = _H * _W


def _body(bias_ref, out_ref, slab_ref, sem):
    hw_r = jax.lax.broadcasted_iota(jnp.int32, (_HW, _HW), 0)
    hw_c = jax.lax.broadcasted_iota(jnp.int32, (_HW, _HW), 1)
    rh = (hw_r >> 4) - (hw_c >> 4)
    rw = (hw_r & 15) - (hw_c & 15)
    p2 = rh * rh + rw * rw
    t = [bias_ref[k, 0] for k in range(6)]
    copies = []
    for delta in range(15):
        s = (p2 + (delta - 7) * (delta - 7)).astype(jnp.float32)
        b = jnp.floor(jnp.sqrt(s) * 0.25)
        slab_ref[delta] = jnp.where(
            b < 1.0, t[0],
            jnp.where(b < 2.0, t[1],
                      jnp.where(b < 3.0, t[2],
                                jnp.where(b < 4.0, t[3],
                                          jnp.where(b < 5.0, t[4], t[5])))))
        for di in range(_D):
            dj = di - (delta - 7)
            if 0 <= dj < _D:
                cp = pltpu.make_async_copy(
                    slab_ref.at[delta],
                    out_ref.at[pl.ds(di * _HW, _HW), pl.ds(dj * _HW, _HW)],
                    sem)
                cp.start()
                copies.append(cp)
    for cp in copies:
        cp.wait()


def kernel(D, H, W, rel_pos_bias):
    del D, H, W
    return pl.pallas_call(
        _body,
        in_specs=[pl.BlockSpec(memory_space=pltpu.VMEM)],
        out_specs=pl.BlockSpec(memory_space=pl.ANY),
        out_shape=jax.ShapeDtypeStruct((_N, _N), jnp.float32),
        scratch_shapes=[
            pltpu.VMEM((15, _HW, _HW), jnp.float32),
            pltpu.SemaphoreType.DMA,
        ],
    )(rel_pos_bias)


# speedup vs baseline: 63.3736x; 1.0066x over previous
import jax
import jax.numpy as jnp
from jax.experimental import pallas as pl
from jax.experimental.pallas import tpu as pltpu

_D, _H, _W = 8, 16, 16
_N = _D * _H * _W
_HW = _H * _W


def _body(bias_ref, out_ref, slab_ref, sem):
    hw_r = jax.lax.broadcasted_iota(jnp.int32, (_HW, _HW), 0)
    hw_c = jax.lax.broadcasted_iota(jnp.int32, (_HW, _HW), 1)
    rh = (hw_r >> 4) - (hw_c >> 4)
    rw = (hw_r & 15) - (hw_c & 15)
    p2 = rh * rh + rw * rw
    t = [bias_ref[k, 0] for k in range(6)]
    copies = []
    for delta in sorted(range(15), key=lambda d: abs(d - 7)):
        s = (p2 + (delta - 7) * (delta - 7)).astype(jnp.float32)
        b = jnp.floor(jnp.sqrt(s) * 0.25)
        slab_ref[delta] = jnp.where(
            b < 1.0, t[0],
            jnp.where(b < 2.0, t[1],
                      jnp.where(b < 3.0, t[2],
                                jnp.where(b < 4.0, t[3],
                                          jnp.where(b < 5.0, t[4], t[5])))))
        for di in range(_D):
            dj = di - (delta - 7)
            if 0 <= dj < _D:
                cp = pltpu.make_async_copy(
                    slab_ref.at[delta],
                    out_ref.at[pl.ds(di * _HW, _HW), pl.ds(dj * _HW, _HW)],
                    sem)
                cp.start()
                copies.append(cp)
    for cp in copies:
        cp.wait()


def kernel(D, H, W, rel_pos_bias):
    del D, H, W
    return pl.pallas_call(
        _body,
        in_specs=[pl.BlockSpec(memory_space=pltpu.VMEM)],
        out_specs=pl.BlockSpec(memory_space=pl.ANY),
        out_shape=jax.ShapeDtypeStruct((_N, _N), jnp.float32),
        scratch_shapes=[
            pltpu.VMEM((15, _HW, _HW), jnp.float32),
            pltpu.SemaphoreType.DMA,
        ],
    )(rel_pos_bias)
